# baseline (speedup 1.0000x reference)
import functools

import jax
import jax.numpy as jnp
from jax.experimental import pallas as pl
from jax.experimental.pallas import tpu as pltpu

_DICE_EPS = 1e-8


def _round_up(x, m):
    return (x + m - 1) // m * m


def _lin_block(h, c, w1h_ref, w1p_ref, w1c_ref, b1_ref):
    return (jnp.dot(h, w1h_ref[...], preferred_element_type=jnp.float32)
            + jnp.dot(h * c, w1p_ref[...], preferred_element_type=jnp.float32)
            + jnp.dot(c, w1c_ref[...], preferred_element_type=jnp.float32)
            + b1_ref[...])


def _lin_stats_kernel(h_ref, c_ref, w1h_ref, w1p_ref, w1c_ref, b1_ref,
                      lint_ref, part_ref, *, batch, tile_rows):
    i = pl.program_id(0)
    h = h_ref[...].astype(jnp.float32)
    c = c_ref[...].astype(jnp.float32)
    lin = _lin_block(h, c, w1h_ref, w1p_ref, w1c_ref, b1_ref)
    lint_ref[0] = lin.T.astype(lint_ref.dtype)

    row_ids = jax.lax.broadcasted_iota(jnp.int32, lin.shape, 0)
    valid = row_ids < (batch - i * tile_rows)
    lin_v = jnp.where(valid, lin, 0.0)
    part_ref[0, 0:1, :] = jnp.sum(lin_v, axis=0, keepdims=True)
    part_ref[0, 1:2, :] = jnp.sum(lin_v * lin_v, axis=0, keepdims=True)


def _gate_project_kernel(part_ref, lint_ref, alpha_ref, w2_ref, b2_ref,
                         out_ref, *, batch):
    inv_b = 1.0 / batch
    s1 = jnp.sum(part_ref[:, 0, :], axis=0, keepdims=True)
    s2 = jnp.sum(part_ref[:, 1, :], axis=0, keepdims=True)
    mean = s1 * inv_b
    var = s2 * inv_b - mean * mean
    rstd = jax.lax.rsqrt(jnp.maximum(var, 0.0) + _DICE_EPS)

    rh = 0.5 * rstd.T
    mh = mean.T * rh
    alpha_c = alpha_ref[...].T
    w2_c = w2_ref[...].T
    u = w2_c * (0.5 * (1.0 + alpha_c))
    v = w2_c * (0.5 * (1.0 - alpha_c))

    lint = lint_ref[0].astype(jnp.float32)
    t = jnp.tanh(lint * rh - mh)
    contrib = lint * (u + v * t)
    logits = jnp.sum(contrib, axis=0, keepdims=True) + b2_ref[0, 0]
    out_ref[...] = logits.reshape(out_ref.shape)


def kernel(history, candidate, w1h, w1p, w1c, b1, alpha, w2, b2,
           *, block_rows=16384):
    B, D = history.shape
    L = b1.shape[-1]

    w1h = w1h.astype(jnp.float32)
    w1p = w1p.astype(jnp.float32)
    w1c = w1c.astype(jnp.float32)
    b1_row = b1.reshape(1, L).astype(jnp.float32)
    alpha_row = jnp.broadcast_to(alpha.astype(jnp.float32), (1, L))
    w2_row = w2.reshape(1, L).astype(jnp.float32)
    b2_s = b2.reshape(1, 1).astype(jnp.float32)

    TB = min(_round_up(block_rows, 8), _round_up(B, 8))
    num_tiles = pl.cdiv(B, TB)
    l_pad = _round_up(L, 128)
    itemsize = jnp.dtype(history.dtype).itemsize

    row_spec = pl.BlockSpec((TB, D), lambda i: (i, 0))
    w1_spec = pl.BlockSpec((D, L), lambda i: (0, 0))
    vecL_spec = pl.BlockSpec((1, L), lambda i: (0, 0))

    p1_bytes = (2 * 2 * TB * D * itemsize
                + 2 * TB * l_pad * 4
                + 3 * D * l_pad * 4 + 4 * l_pad * 4)
    lint_hbm, parts = pl.pallas_call(
        functools.partial(_lin_stats_kernel, batch=B, tile_rows=TB),
        out_shape=(jax.ShapeDtypeStruct((num_tiles, L, TB), jnp.bfloat16),
                   jax.ShapeDtypeStruct((num_tiles, 2, L), jnp.float32)),
        grid=(num_tiles,),
        in_specs=[row_spec, row_spec, w1_spec, w1_spec, w1_spec, vecL_spec],
        out_specs=(pl.BlockSpec((1, L, TB), lambda i: (i, 0, 0)),
                   pl.BlockSpec((1, 2, L), lambda i: (i, 0, 0))),
        compiler_params=pltpu.CompilerParams(
            dimension_semantics=("parallel",),
            vmem_limit_bytes=int(min(60 * 1024 * 1024, p1_bytes * 2))),
    )(history, candidate, w1h, w1p, w1c, b1_row)

    split = 2 if TB % 2 == 0 else 1
    TB2 = TB // split
    nt2 = num_tiles * split
    p2_bytes = (4 * TB2 * _round_up(L, 16) * 2 + num_tiles * 2 * l_pad * 4
                + 4 * l_pad * 4 + 2 * TB2 * 4)
    out = pl.pallas_call(
        functools.partial(_gate_project_kernel, batch=B),
        out_shape=jax.ShapeDtypeStruct((nt2, 1, TB2), jnp.float32),
        grid=(nt2,),
        in_specs=[
            pl.BlockSpec((num_tiles, 2, L), lambda i: (0, 0, 0)),
            pl.BlockSpec((1, L, TB2), lambda i: (i // split, 0, i % split)),
            vecL_spec, vecL_spec,
            pl.BlockSpec(memory_space=pltpu.MemorySpace.SMEM),
        ],
        out_specs=pl.BlockSpec((1, 1, TB2), lambda i: (i, 0, 0)),
        compiler_params=pltpu.CompilerParams(
            dimension_semantics=("parallel",),
            vmem_limit_bytes=int(min(60 * 1024 * 1024, p2_bytes * 4))),
    )(parts, lint_hbm, alpha_row, w2_row, b2_s)

    return out.reshape(nt2 * TB2)[:B].reshape(B, 1)


# speedup vs baseline: 1.0717x; 1.0717x over previous
import functools

import jax
import jax.numpy as jnp
from jax.experimental import pallas as pl
from jax.experimental.pallas import tpu as pltpu

_DICE_EPS = 1e-8


def _round_up(x, m):
    return (x + m - 1) // m * m


def _lin_block(h, c, w1h_ref, w1p_ref, w1c_ref, b1_ref):
    return (jnp.dot(h, w1h_ref[...], preferred_element_type=jnp.float32)
            + jnp.dot(h * c, w1p_ref[...], preferred_element_type=jnp.float32)
            + jnp.dot(c, w1c_ref[...], preferred_element_type=jnp.float32)
            + b1_ref[...])


def _lin_stats_kernel(h_ref, c_ref, w1h_ref, w1p_ref, w1c_ref, b1_ref,
                      lint_ref, part_ref, *, batch, tile_rows):
    i = pl.program_id(0)
    h = h_ref[...].astype(jnp.float32)
    c = c_ref[...].astype(jnp.float32)
    lin = _lin_block(h, c, w1h_ref, w1p_ref, w1c_ref, b1_ref)
    lint_ref[0] = lin.T.astype(lint_ref.dtype)

    row_ids = jax.lax.broadcasted_iota(jnp.int32, lin.shape, 0)
    valid = row_ids < (batch - i * tile_rows)
    lin_v = jnp.where(valid, lin, 0.0)
    part_ref[0, 0:1, :] = jnp.sum(lin_v, axis=0, keepdims=True)
    part_ref[0, 1:2, :] = jnp.sum(lin_v * lin_v, axis=0, keepdims=True)


def _gate_project_kernel(part_ref, lint_ref, alpha_ref, w2_ref, b2_ref,
                         out_ref, *, batch):
    inv_b = 1.0 / batch
    s1 = jnp.sum(part_ref[:, 0, :], axis=0, keepdims=True)
    s2 = jnp.sum(part_ref[:, 1, :], axis=0, keepdims=True)
    mean = s1 * inv_b
    var = s2 * inv_b - mean * mean
    rstd = jax.lax.rsqrt(jnp.maximum(var, 0.0) + _DICE_EPS)

    rh = 0.5 * rstd.T
    mh = mean.T * rh
    alpha_c = alpha_ref[...].T
    w2_c = w2_ref[...].T
    u = w2_c * (0.5 * (1.0 + alpha_c))
    v = w2_c * (0.5 * (1.0 - alpha_c))

    lint = lint_ref[0].astype(jnp.float32)
    t = jnp.tanh(lint * rh - mh)
    s = lint * t
    logits = (jnp.dot(u.T, lint, preferred_element_type=jnp.float32)
              + jnp.dot(v.T, s, preferred_element_type=jnp.float32)
              + b2_ref[0, 0])
    out_ref[...] = logits.reshape(out_ref.shape)


def kernel(history, candidate, w1h, w1p, w1c, b1, alpha, w2, b2,
           *, block_rows=16384):
    B, D = history.shape
    L = b1.shape[-1]

    w1h = w1h.astype(jnp.float32)
    w1p = w1p.astype(jnp.float32)
    w1c = w1c.astype(jnp.float32)
    b1_row = b1.reshape(1, L).astype(jnp.float32)
    alpha_row = jnp.broadcast_to(alpha.astype(jnp.float32), (1, L))
    w2_row = w2.reshape(1, L).astype(jnp.float32)
    b2_s = b2.reshape(1, 1).astype(jnp.float32)

    TB = min(_round_up(block_rows, 8), _round_up(B, 8))
    num_tiles = pl.cdiv(B, TB)
    l_pad = _round_up(L, 128)
    itemsize = jnp.dtype(history.dtype).itemsize

    row_spec = pl.BlockSpec((TB, D), lambda i: (i, 0))
    w1_spec = pl.BlockSpec((D, L), lambda i: (0, 0))
    vecL_spec = pl.BlockSpec((1, L), lambda i: (0, 0))

    p1_bytes = (2 * 2 * TB * D * itemsize
                + 2 * TB * l_pad * 4
                + 3 * D * l_pad * 4 + 4 * l_pad * 4)
    lint_hbm, parts = pl.pallas_call(
        functools.partial(_lin_stats_kernel, batch=B, tile_rows=TB),
        out_shape=(jax.ShapeDtypeStruct((num_tiles, L, TB), jnp.bfloat16),
                   jax.ShapeDtypeStruct((num_tiles, 2, L), jnp.float32)),
        grid=(num_tiles,),
        in_specs=[row_spec, row_spec, w1_spec, w1_spec, w1_spec, vecL_spec],
        out_specs=(pl.BlockSpec((1, L, TB), lambda i: (i, 0, 0)),
                   pl.BlockSpec((1, 2, L), lambda i: (i, 0, 0))),
        compiler_params=pltpu.CompilerParams(
            dimension_semantics=("parallel",),
            vmem_limit_bytes=int(min(60 * 1024 * 1024, p1_bytes * 2))),
    )(history, candidate, w1h, w1p, w1c, b1_row)

    split = 1
    TB2 = TB // split
    nt2 = num_tiles * split
    p2_bytes = (4 * TB2 * _round_up(L, 16) * 2 + num_tiles * 2 * l_pad * 4
                + 4 * l_pad * 4 + 2 * TB2 * 4)
    out = pl.pallas_call(
        functools.partial(_gate_project_kernel, batch=B),
        out_shape=jax.ShapeDtypeStruct((nt2, 1, TB2), jnp.float32),
        grid=(nt2,),
        in_specs=[
            pl.BlockSpec((num_tiles, 2, L), lambda i: (0, 0, 0)),
            pl.BlockSpec((1, L, TB2), lambda i: (i // split, 0, i % split)),
            vecL_spec, vecL_spec,
            pl.BlockSpec(memory_space=pltpu.MemorySpace.SMEM),
        ],
        out_specs=pl.BlockSpec((1, 1, TB2), lambda i: (i, 0, 0)),
        compiler_params=pltpu.CompilerParams(
            dimension_semantics=("parallel",),
            vmem_limit_bytes=int(min(60 * 1024 * 1024, p2_bytes * 4))),
    )(parts, lint_hbm, alpha_row, w2_row, b2_s)

    return out.reshape(nt2 * TB2)[:B].reshape(B, 1)


# speedup vs baseline: 1.1129x; 1.0385x over previous
import functools

import jax
import jax.numpy as jnp
from jax.experimental import pallas as pl
from jax.experimental.pallas import tpu as pltpu

_DICE_EPS = 1e-8


def _round_up(x, m):
    return (x + m - 1) // m * m


def _lin_block(h, c, w1h_ref, w1p_ref, w1c_ref, b1_ref):
    return (jnp.dot(h, w1h_ref[...], preferred_element_type=jnp.float32)
            + jnp.dot(h * c, w1p_ref[...], preferred_element_type=jnp.float32)
            + jnp.dot(c, w1c_ref[...], preferred_element_type=jnp.float32)
            + b1_ref[...])


def _lin_stats_kernel(h_ref, c_ref, w1h_ref, w1p_ref, w1c_ref, b1_ref,
                      lint_ref, part_ref, *, batch, tile_rows):
    i = pl.program_id(0)
    h = h_ref[...].astype(jnp.float32)
    c = c_ref[...].astype(jnp.float32)
    lin = _lin_block(h, c, w1h_ref, w1p_ref, w1c_ref, b1_ref)
    lint = lin.T
    lint_ref[0] = lint.astype(lint_ref.dtype)

    col_ids = jax.lax.broadcasted_iota(jnp.int32, (1, lint.shape[1]), 1)
    valid = col_ids < (batch - i * tile_rows)
    lint_v = jnp.where(valid, lint, 0.0)
    part_ref[0, :, 0:1] = jnp.sum(lint_v, axis=1, keepdims=True)
    part_ref[0, :, 1:2] = jnp.sum(lint_v * lint_v, axis=1, keepdims=True)


def _gate_project_kernel(part_ref, lint_ref, alpha_ref, w2_ref, b2_ref,
                         out_ref, *, batch):
    inv_b = 1.0 / batch
    s1 = jnp.sum(part_ref[:, :, 0:1], axis=0)
    s2 = jnp.sum(part_ref[:, :, 1:2], axis=0)
    mean = s1 * inv_b
    var = s2 * inv_b - mean * mean
    rstd = jax.lax.rsqrt(jnp.maximum(var, 0.0) + _DICE_EPS)

    rh = 0.5 * rstd
    mh = mean * rh
    alpha_c = alpha_ref[...].T
    w2_c = w2_ref[...].T
    u = w2_c * (0.5 * (1.0 + alpha_c))
    v = w2_c * (0.5 * (1.0 - alpha_c))

    lint = lint_ref[0].astype(jnp.float32)
    t = jnp.tanh(lint * rh - mh)
    s = lint * t
    logits = (jnp.dot(u.T, lint, preferred_element_type=jnp.float32)
              + jnp.dot(v.T, s, preferred_element_type=jnp.float32)
              + b2_ref[0, 0])
    out_ref[...] = logits.reshape(out_ref.shape)


def kernel(history, candidate, w1h, w1p, w1c, b1, alpha, w2, b2,
           *, block_rows=16384):
    B, D = history.shape
    L = b1.shape[-1]

    w1h = w1h.astype(jnp.float32)
    w1p = w1p.astype(jnp.float32)
    w1c = w1c.astype(jnp.float32)
    b1_row = b1.reshape(1, L).astype(jnp.float32)
    alpha_row = jnp.broadcast_to(alpha.astype(jnp.float32), (1, L))
    w2_row = w2.reshape(1, L).astype(jnp.float32)
    b2_s = b2.reshape(1, 1).astype(jnp.float32)

    TB = min(_round_up(block_rows, 8), _round_up(B, 8))
    num_tiles = pl.cdiv(B, TB)
    l_pad = _round_up(L, 128)
    itemsize = jnp.dtype(history.dtype).itemsize

    row_spec = pl.BlockSpec((TB, D), lambda i: (i, 0))
    w1_spec = pl.BlockSpec((D, L), lambda i: (0, 0))
    vecL_spec = pl.BlockSpec((1, L), lambda i: (0, 0))

    p1_bytes = (2 * 2 * TB * D * itemsize
                + 2 * TB * l_pad * 4
                + 3 * D * l_pad * 4 + 4 * l_pad * 4)
    lint_hbm, parts = pl.pallas_call(
        functools.partial(_lin_stats_kernel, batch=B, tile_rows=TB),
        out_shape=(jax.ShapeDtypeStruct((num_tiles, L, TB), jnp.bfloat16),
                   jax.ShapeDtypeStruct((num_tiles, L, 2), jnp.float32)),
        grid=(num_tiles,),
        in_specs=[row_spec, row_spec, w1_spec, w1_spec, w1_spec, vecL_spec],
        out_specs=(pl.BlockSpec((1, L, TB), lambda i: (i, 0, 0)),
                   pl.BlockSpec((1, L, 2), lambda i: (i, 0, 0))),
        compiler_params=pltpu.CompilerParams(
            dimension_semantics=("parallel",),
            vmem_limit_bytes=int(min(60 * 1024 * 1024, p1_bytes * 2))),
    )(history, candidate, w1h, w1p, w1c, b1_row)

    split = 1
    TB2 = TB // split
    nt2 = num_tiles * split
    p2_bytes = (4 * TB2 * _round_up(L, 16) * 2 + num_tiles * 2 * l_pad * 4
                + 4 * l_pad * 4 + 2 * TB2 * 4)
    out = pl.pallas_call(
        functools.partial(_gate_project_kernel, batch=B),
        out_shape=jax.ShapeDtypeStruct((nt2, 1, TB2), jnp.float32),
        grid=(nt2,),
        in_specs=[
            pl.BlockSpec((num_tiles, L, 2), lambda i: (0, 0, 0)),
            pl.BlockSpec((1, L, TB2), lambda i: (i // split, 0, i % split)),
            vecL_spec, vecL_spec,
            pl.BlockSpec(memory_space=pltpu.MemorySpace.SMEM),
        ],
        out_specs=pl.BlockSpec((1, 1, TB2), lambda i: (i, 0, 0)),
        compiler_params=pltpu.CompilerParams(
            dimension_semantics=("parallel",),
            vmem_limit_bytes=int(min(60 * 1024 * 1024, p2_bytes * 4))),
    )(parts, lint_hbm, alpha_row, w2_row, b2_s)

    return out.reshape(nt2 * TB2)[:B].reshape(B, 1)
